# baseline (speedup 1.0000x reference)
import functools

import jax
import jax.numpy as jnp
import numpy as np
from jax import lax
from jax.experimental import pallas as pl
from jax.experimental.pallas import tpu as pltpu
from jax.experimental.pallas import tpu_sc as plsc

_NC, _NS, _L = 2, 16, 16
_NW = _NC * _NS
_C = 19
_BINS = _C * _C
_HSTR = 369
_HLEN = _HSTR * _L
_CHUNKS = 2
_NHIST = 4
_W = 512


def _argmax_body(tgt_ref, x_ref, idx_ref):
    x = x_ref[0]
    t = tgt_ref[0]
    m = x[0]
    p = jnp.zeros_like(t)
    for c in range(1, _C):
        xc = x[c]
        g = xc > m
        m = jnp.where(g, xc, m)
        p = jnp.where(g, c, p)
    idx_ref[0] = t * _C + p


def _bincount_body(rows, wpb, idx_hbm, out_hbm, chunk_v, hists, sem):
    wid = lax.axis_index("s") * _NC + lax.axis_index("c")
    b = wid // wpb
    r0 = (wid % wpb) * rows
    zeros16 = jnp.zeros((_L,), jnp.float32)
    ones16 = jnp.ones((_L,), jnp.float32)
    lane_base = lax.iota(jnp.int32, _L) * _HSTR

    cp = pltpu.async_copy(idx_hbm.at[b, pl.ds(r0, rows)], chunk_v, sem)

    def zero_body(i, carry):
        for h in hists:
            h[pl.ds(i * _L, _L)] = zeros16
        return carry

    lax.fori_loop(0, _HLEN // _L, zero_body, 0)
    cp.wait()

    def body(i, carry):
        col = i * _L
        for r in range(rows):
            v = chunk_v[r, pl.ds(col, _L)]
            si = lane_base + v
            plsc.addupdate_scatter(hists[r % _NHIST], [si], ones16)
        return carry

    lax.fori_loop(0, _W // _L, body, 0)

    def merge_body(i, carry):
        s = i * _L
        acc = hists[0][pl.ds(s, _L)]
        for h in hists[1:]:
            acc = acc + h[pl.ds(s, _L)]
        hists[0][pl.ds(s, _L)] = acc
        return carry

    lax.fori_loop(0, _HLEN // _L, merge_body, 0)

    pltpu.sync_copy(hists[0], out_hbm.at[wid])


def _bincount_wrapper(rows, wpb, idx_hbm, out_hbm, chunk_v, *rest):
    hists, sem = rest[:-1], rest[-1]
    _bincount_body(rows, wpb, idx_hbm, out_hbm, chunk_v, hists, sem)


def _epilogue_body(n_total, *refs):
    (*part_refs, mrow_ref, mcol_ref, mdiag_ref, loss_ref) = refs
    flat = jnp.sum(part_refs[0][...], axis=0)
    for pr in part_refs[1:]:
        flat = flat + jnp.sum(pr[...], axis=0)
    f = flat[None, :]
    ht = jnp.sum(mrow_ref[...] * f, axis=1)
    hp = jnp.sum(mcol_ref[...] * f, axis=1)
    tp = jnp.sum(mdiag_ref[...] * f, axis=1)
    smooth = jnp.float32(1e-6)
    ntot = jnp.float32(n_total)
    sens = (tp + smooth) / (hp + smooth)
    spec = (ntot - ht - hp + tp + smooth) / (ntot - hp + smooth)
    mean = jnp.sum(0.5 * sens + 0.5 * spec) / _C
    loss_ref[...] = 1.0 - mean.reshape(1, 1)


def _masks():
    j = np.arange(_HLEN) % _HSTR
    t, p = j // _C, j % _C
    valid = j < _BINS
    r = np.arange(_C)[:, None]
    mrow = ((t[None, :] == r) & valid[None, :]).astype(np.float32)
    mcol = ((p[None, :] == r) & valid[None, :]).astype(np.float32)
    mdiag = ((j[None, :] == 20 * r) & valid[None, :]).astype(np.float32)
    return mrow, mcol, mdiag


_MROW, _MCOL, _MDIAG = _masks()


def kernel(output, target):
    B, C, H, W = output.shape
    Hb = 512
    n_total = B * H * W
    nb = B // _CHUNKS
    wpb = _NW // nb
    rows = H // wpb

    sc_bincount = functools.partial(
        pl.kernel,
        mesh=plsc.VectorSubcoreMesh(core_axis_name="c", subcore_axis_name="s"),
        out_type=jax.ShapeDtypeStruct((_NW, _HLEN), jnp.float32),
        scratch_types=[
            pltpu.VMEM((rows, W), jnp.int32),
        ] + [pltpu.VMEM((_HLEN,), jnp.float32)] * _NHIST
          + [pltpu.SemaphoreType.DMA],
        compiler_params=pltpu.CompilerParams(
            needs_layout_passes=False, disable_bounds_checks=True,
        ),
    )(functools.partial(_bincount_wrapper, rows, wpb))

    parts = []
    for k in range(_CHUNKS):
        idx_k = pl.pallas_call(
            _argmax_body,
            grid=(nb, H // Hb),
            in_specs=[
                pl.BlockSpec((1, Hb, W),
                             lambda b, r, k=k: (b + k * nb, r, 0)),
                pl.BlockSpec((1, C, Hb, W),
                             lambda b, r, k=k: (b + k * nb, 0, r, 0)),
            ],
            out_specs=pl.BlockSpec((1, Hb, W), lambda b, r: (b, r, 0)),
            out_shape=jax.ShapeDtypeStruct((nb, H, W), jnp.int32),
            compiler_params=pltpu.CompilerParams(
                dimension_semantics=("parallel", "parallel"),
            ),
        )(target, output)
        parts.append(sc_bincount(idx_k))

    loss = pl.pallas_call(
        functools.partial(_epilogue_body, n_total),
        out_shape=jax.ShapeDtypeStruct((1, 1), jnp.float32),
    )(*parts, _MROW, _MCOL, _MDIAG)
    return loss[0, 0]


# speedup vs baseline: 1.1289x; 1.1289x over previous
import functools

import jax
import jax.numpy as jnp
import numpy as np
from jax import lax
from jax.experimental import pallas as pl
from jax.experimental.pallas import tpu as pltpu
from jax.experimental.pallas import tpu_sc as plsc

_NC, _NS, _L = 2, 16, 16
_NW = _NC * _NS
_C = 19
_BINS = _C * _C
_HSTR = 369
_HLEN = _HSTR * _L
_CHUNKS = 2
_NHIST = 4
_W = 512


def _argmax_body(tgt_ref, x_ref, idx_ref):
    x = x_ref[0]
    t = tgt_ref[0]
    m = x[0]
    p = jnp.zeros_like(t)
    for c in range(1, _C):
        xc = x[c]
        g = xc > m
        m = jnp.where(g, xc, m)
        p = jnp.where(g, c, p)
    idx_ref[0] = t * _C + p


def _bincount_body(rows, wpb, idx_hbm, out_hbm, chunk_v, hists, sem):
    wid = lax.axis_index("s") * _NC + lax.axis_index("c")
    b = wid // wpb
    r0 = (wid % wpb) * rows
    zeros16 = jnp.zeros((_L,), jnp.float32)
    ones16 = jnp.ones((_L,), jnp.float32)
    lane_base = lax.iota(jnp.int32, _L) * _HSTR

    cp = pltpu.async_copy(idx_hbm.at[b, pl.ds(r0, rows)], chunk_v, sem)

    def zero_body(i, carry):
        for h in hists:
            h[pl.ds(i * _L, _L)] = zeros16
        return carry

    lax.fori_loop(0, _HLEN // _L, zero_body, 0)
    cp.wait()

    @plsc.parallel_loop(0, _W // _L, 1, unroll=2)
    def body(i):
        col = i * _L
        for r in range(rows):
            v = chunk_v[r, pl.ds(col, _L)]
            si = lane_base + v
            plsc.addupdate_scatter(hists[r % _NHIST], [si], ones16)

    def merge_body(i, carry):
        s = i * _L
        acc = hists[0][pl.ds(s, _L)]
        for h in hists[1:]:
            acc = acc + h[pl.ds(s, _L)]
        hists[0][pl.ds(s, _L)] = acc
        return carry

    lax.fori_loop(0, _HLEN // _L, merge_body, 0)

    pltpu.sync_copy(hists[0], out_hbm.at[wid])


def _bincount_wrapper(rows, wpb, idx_hbm, out_hbm, chunk_v, *rest):
    hists, sem = rest[:-1], rest[-1]
    _bincount_body(rows, wpb, idx_hbm, out_hbm, chunk_v, hists, sem)


def _epilogue_body(n_total, *refs):
    (*part_refs, mrow_ref, mcol_ref, mdiag_ref, loss_ref) = refs
    flat = jnp.sum(part_refs[0][...], axis=0)
    for pr in part_refs[1:]:
        flat = flat + jnp.sum(pr[...], axis=0)
    f = flat[None, :]
    ht = jnp.sum(mrow_ref[...] * f, axis=1)
    hp = jnp.sum(mcol_ref[...] * f, axis=1)
    tp = jnp.sum(mdiag_ref[...] * f, axis=1)
    smooth = jnp.float32(1e-6)
    ntot = jnp.float32(n_total)
    sens = (tp + smooth) / (hp + smooth)
    spec = (ntot - ht - hp + tp + smooth) / (ntot - hp + smooth)
    mean = jnp.sum(0.5 * sens + 0.5 * spec) / _C
    loss_ref[...] = 1.0 - mean.reshape(1, 1)


def _masks():
    j = np.arange(_HLEN) % _HSTR
    t, p = j // _C, j % _C
    valid = j < _BINS
    r = np.arange(_C)[:, None]
    mrow = ((t[None, :] == r) & valid[None, :]).astype(np.float32)
    mcol = ((p[None, :] == r) & valid[None, :]).astype(np.float32)
    mdiag = ((j[None, :] == 20 * r) & valid[None, :]).astype(np.float32)
    return mrow, mcol, mdiag


_MROW, _MCOL, _MDIAG = _masks()


def kernel(output, target):
    B, C, H, W = output.shape
    Hb = 256
    n_total = B * H * W
    nb = B // _CHUNKS
    wpb = _NW // nb
    rows = H // wpb

    sc_bincount = functools.partial(
        pl.kernel,
        mesh=plsc.VectorSubcoreMesh(core_axis_name="c", subcore_axis_name="s"),
        out_type=jax.ShapeDtypeStruct((_NW, _HLEN), jnp.float32),
        scratch_types=[
            pltpu.VMEM((rows, W), jnp.int32),
        ] + [pltpu.VMEM((_HLEN,), jnp.float32)] * _NHIST
          + [pltpu.SemaphoreType.DMA],
        compiler_params=pltpu.CompilerParams(
            needs_layout_passes=False, disable_bounds_checks=True,
        ),
    )(functools.partial(_bincount_wrapper, rows, wpb))

    parts = []
    for k in range(_CHUNKS):
        idx_k = pl.pallas_call(
            _argmax_body,
            grid=(nb, H // Hb),
            in_specs=[
                pl.BlockSpec((1, Hb, W),
                             lambda b, r, k=k: (b + k * nb, r, 0)),
                pl.BlockSpec((1, C, Hb, W),
                             lambda b, r, k=k: (b + k * nb, 0, r, 0)),
            ],
            out_specs=pl.BlockSpec((1, Hb, W), lambda b, r: (b, r, 0)),
            out_shape=jax.ShapeDtypeStruct((nb, H, W), jnp.int32),
            compiler_params=pltpu.CompilerParams(
                dimension_semantics=("parallel", "parallel"),
            ),
        )(target, output)
        parts.append(sc_bincount(idx_k))

    loss = pl.pallas_call(
        functools.partial(_epilogue_body, n_total),
        out_shape=jax.ShapeDtypeStruct((1, 1), jnp.float32),
    )(*parts, _MROW, _MCOL, _MDIAG)
    return loss[0, 0]


# speedup vs baseline: 1.1555x; 1.0235x over previous
import functools

import jax
import jax.numpy as jnp
import numpy as np
from jax import lax
from jax.experimental import pallas as pl
from jax.experimental.pallas import tpu as pltpu
from jax.experimental.pallas import tpu_sc as plsc

_NC, _NS, _L = 2, 16, 16
_NW = _NC * _NS
_C = 19
_BINS = _C * _C
_HSTR = 369
_HLEN = _HSTR * _L
_CHUNKS = 2
_NHIST = 4
_W = 512


def _argmax_body(tgt_ref, x_ref, idx_ref):
    x = x_ref[0]
    t = tgt_ref[0]
    m = x[0]
    p = jnp.zeros_like(t)
    for c in range(1, _C):
        xc = x[c]
        g = xc > m
        m = jnp.where(g, xc, m)
        p = jnp.where(g, c, p)
    idx_ref[0] = t * _C + p


def _bincount_body(rows, wpb, idx_hbm, out_hbm, chunk_v, hists, sem):
    wid = lax.axis_index("s") * _NC + lax.axis_index("c")
    b = wid // wpb
    r0 = (wid % wpb) * rows
    zeros16 = jnp.zeros((_L,), jnp.float32)
    ones16 = jnp.ones((_L,), jnp.float32)
    lane_base = lax.iota(jnp.int32, _L) * _HSTR

    cp = pltpu.async_copy(idx_hbm.at[b, pl.ds(r0, rows)], chunk_v, sem)

    @plsc.parallel_loop(0, _HLEN // _L, 1, unroll=4)
    def zero_body(i):
        for h in hists:
            h[pl.ds(i * _L, _L)] = zeros16

    cp.wait()

    @plsc.parallel_loop(0, _W // _L, 1, unroll=2)
    def body(i):
        col = i * _L
        for r in range(rows):
            v = chunk_v[r, pl.ds(col, _L)]
            si = lane_base + v
            plsc.addupdate_scatter(hists[r % _NHIST], [si], ones16)

    @plsc.parallel_loop(0, _HLEN // _L, 1, unroll=4)
    def merge_body(i):
        s = i * _L
        acc = hists[0][pl.ds(s, _L)]
        for h in hists[1:]:
            acc = acc + h[pl.ds(s, _L)]
        hists[0][pl.ds(s, _L)] = acc

    pltpu.sync_copy(hists[0], out_hbm.at[wid])


def _bincount_wrapper(rows, wpb, idx_hbm, out_hbm, chunk_v, *rest):
    hists, sem = rest[:-1], rest[-1]
    _bincount_body(rows, wpb, idx_hbm, out_hbm, chunk_v, hists, sem)


def _epilogue_body(n_total, *refs):
    (*part_refs, mrow_ref, mcol_ref, mdiag_ref, loss_ref) = refs
    flat = jnp.sum(part_refs[0][...], axis=0)
    for pr in part_refs[1:]:
        flat = flat + jnp.sum(pr[...], axis=0)
    f = flat[None, :]
    ht = jnp.sum(mrow_ref[...] * f, axis=1)
    hp = jnp.sum(mcol_ref[...] * f, axis=1)
    tp = jnp.sum(mdiag_ref[...] * f, axis=1)
    smooth = jnp.float32(1e-6)
    ntot = jnp.float32(n_total)
    sens = (tp + smooth) / (hp + smooth)
    spec = (ntot - ht - hp + tp + smooth) / (ntot - hp + smooth)
    mean = jnp.sum(0.5 * sens + 0.5 * spec) / _C
    loss_ref[...] = 1.0 - mean.reshape(1, 1)


def _masks():
    j = np.arange(_HLEN) % _HSTR
    t, p = j // _C, j % _C
    valid = j < _BINS
    r = np.arange(_C)[:, None]
    mrow = ((t[None, :] == r) & valid[None, :]).astype(np.float32)
    mcol = ((p[None, :] == r) & valid[None, :]).astype(np.float32)
    mdiag = ((j[None, :] == 20 * r) & valid[None, :]).astype(np.float32)
    return mrow, mcol, mdiag


_MROW, _MCOL, _MDIAG = _masks()


def kernel(output, target):
    B, C, H, W = output.shape
    Hb = 256
    n_total = B * H * W
    nb = B // _CHUNKS
    wpb = _NW // nb
    rows = H // wpb

    sc_bincount = functools.partial(
        pl.kernel,
        mesh=plsc.VectorSubcoreMesh(core_axis_name="c", subcore_axis_name="s"),
        out_type=jax.ShapeDtypeStruct((_NW, _HLEN), jnp.float32),
        scratch_types=[
            pltpu.VMEM((rows, W), jnp.int32),
        ] + [pltpu.VMEM((_HLEN,), jnp.float32)] * _NHIST
          + [pltpu.SemaphoreType.DMA],
        compiler_params=pltpu.CompilerParams(
            needs_layout_passes=False, disable_bounds_checks=True,
        ),
    )(functools.partial(_bincount_wrapper, rows, wpb))

    parts = []
    for k in range(_CHUNKS):
        idx_k = pl.pallas_call(
            _argmax_body,
            grid=(nb, H // Hb),
            in_specs=[
                pl.BlockSpec((1, Hb, W),
                             lambda b, r, k=k: (b + k * nb, r, 0)),
                pl.BlockSpec((1, C, Hb, W),
                             lambda b, r, k=k: (b + k * nb, 0, r, 0)),
            ],
            out_specs=pl.BlockSpec((1, Hb, W), lambda b, r: (b, r, 0)),
            out_shape=jax.ShapeDtypeStruct((nb, H, W), jnp.int32),
            compiler_params=pltpu.CompilerParams(
                dimension_semantics=("parallel", "parallel"),
            ),
        )(target, output)
        parts.append(sc_bincount(idx_k))

    loss = pl.pallas_call(
        functools.partial(_epilogue_body, n_total),
        out_shape=jax.ShapeDtypeStruct((1, 1), jnp.float32),
    )(*parts, _MROW, _MCOL, _MDIAG)
    return loss[0, 0]
